# baseline (speedup 1.0000x reference)
import functools

import jax
import jax.numpy as jnp
from jax import lax
from jax.experimental import pallas as pl
from jax.experimental.pallas import tpu as pltpu
from jax.experimental.pallas import tpu_sc as plsc

NC = 2
NS = 16
L = 16
NW = NC * NS
B = 16384
D = 64
N = 100000
BPW = B // NW
NCH = BPW // 128
HALF = BPW // 2
CPH = HALF // L
SR = N * D // 128
NBLK = N // 128
BPT = (NBLK + NW - 1) // NW


TW = 8192
TH = TW // 2
NTB = (N + TW - 1) // TW
SRP = NTB * TH


def _tstage_body(inT_ref, out_ref):
    x = inT_ref[...]
    out_ref[:, 0:D] = x[:, 0:TH].T
    out_ref[:, D:128] = x[:, TH:TW].T


def _tstage(tableT):
    return pl.pallas_call(
        _tstage_body,
        grid=(NTB,),
        in_specs=[pl.BlockSpec((D, TW), lambda j: (0, j))],
        out_specs=pl.BlockSpec((TH, 128), lambda j: (j, 0)),
        out_shape=jax.ShapeDtypeStruct((SRP, 128), jnp.float32),
    )(tableT)


def _svdpp_body(sids_h, pids_h, sfac_h, pfac_h, sbias_h, pbias_h, g_h, out_h,
                sidx_v, pidx_v, sdidx_v, pdidx_v, sh_v, ph_v,
                srows_v, prows_v, sb_v, pb_v, g_v, out_v, sem, semb):
    w = lax.axis_index("s") * NC + lax.axis_index("c")
    base = w * BPW

    pltpu.sync_copy(sids_h.at[pl.ds(w * NCH, NCH)], sidx_v)
    pltpu.sync_copy(pids_h.at[pl.ds(w * NCH, NCH)], pidx_v)
    pltpu.sync_copy(g_h, g_v)

    bcps = []
    for c in range(NCH):
        r = pl.ds(c * 128, 128)
        bcps.append(pltpu.async_copy(sbias_h.at[sidx_v.at[c]], sb_v.at[r], semb))
        bcps.append(pltpu.async_copy(pbias_h.at[pidx_v.at[c]], pb_v.at[r], semb))

    sh1 = TW.bit_length() - 1
    sh2 = sh1 - 1

    def _didx(v):
        return lax.shift_left(lax.shift_right_logical(v, sh1), sh2) + \
            lax.bitwise_and(v, TW // 2 - 1)

    def _hoff(v):
        return lax.shift_left(
            lax.bitwise_and(lax.shift_right_logical(v, sh2), 1), 6)

    for c in range(NCH):
        for j in range(128 // L):
            sl = pl.ds(j * L, L)
            fl = pl.ds(c * 128 + j * L, L)
            sv = sidx_v[c, sl]
            pv = pidx_v[c, sl]
            sdidx_v[c, sl] = _didx(sv)
            pdidx_v[c, sl] = _didx(pv)
            sh_v[fl] = _hoff(sv)
            ph_v[fl] = _hoff(pv)

    iota = lax.broadcasted_iota(jnp.int32, (L,), 0)
    ones = jnp.ones((L,), jnp.int32)
    gvec = g_v[...]

    sbufs = (srows_v.at[pl.ds(0, 128)], srows_v.at[pl.ds(128, 128)])
    pbufs = (prows_v.at[pl.ds(0, 128)], prows_v.at[pl.ds(128, 128)])

    def fire(q):
        return [pltpu.async_copy(sfac_h.at[sdidx_v.at[q]], sbufs[q % 2], sem),
                pltpu.async_copy(pfac_h.at[pdidx_v.at[q]], pbufs[q % 2], sem)]

    pend = fire(0)
    for q in range(NCH):
        if q + 1 < NCH:
            nxt = fire(q + 1)
        for cp in pend:
            cp.wait()
        if q == 0:
            for cp in bcps:
                cp.wait()
        def chunk(i, carry, q=q):
            lrow = (q % 2) * 128 + (i - q * 8) * L + iota
            sh0 = sh_v[pl.ds(i * L, L)]
            ph0 = ph_v[pl.ds(i * L, L)]
            z = jnp.zeros((L,), jnp.float32)

            def dbody(t, st):
                a0, a1, a2, a3, scol, pcol = st
                accs = [a0, a1, a2, a3]
                for k in range(8):
                    sc = scol + k if k else scol
                    pc = pcol + k if k else pcol
                    sv = plsc.load_gather(srows_v, [lrow, sc])
                    pv = plsc.load_gather(prows_v, [lrow, pc])
                    accs[k % 4] = accs[k % 4] + sv * pv
                return (*accs, scol + 8, pcol + 8)

            a0, a1, a2, a3, _, _ = lax.fori_loop(
                0, D // 8, dbody, (z, z, z, z, sh0, ph0))
            sl = pl.ds(i * L, L)
            out_v[sl] = ((a0 + a1) + (a2 + a3)) + sb_v[sl] + pb_v[sl] + gvec
            return carry

        lax.fori_loop(q * 8, (q + 1) * 8, chunk, 0)
        if q + 1 < NCH:
            pend = nxt

    pltpu.sync_copy(out_v, out_h.at[pl.ds(base, BPW)])


_svdpp = functools.partial(
    pl.kernel,
    out_type=jax.ShapeDtypeStruct((B,), jnp.float32),
    mesh=plsc.VectorSubcoreMesh(core_axis_name="c", subcore_axis_name="s"),
    scratch_types=[
        pltpu.VMEM((NCH, 128), jnp.int32),
        pltpu.VMEM((NCH, 128), jnp.int32),
        pltpu.VMEM((NCH, 128), jnp.int32),
        pltpu.VMEM((NCH, 128), jnp.int32),
        pltpu.VMEM((BPW,), jnp.int32),
        pltpu.VMEM((BPW,), jnp.int32),
        pltpu.VMEM((HALF, 128), jnp.float32),
        pltpu.VMEM((HALF, 128), jnp.float32),
        pltpu.VMEM((BPW,), jnp.float32),
        pltpu.VMEM((BPW,), jnp.float32),
        pltpu.VMEM((L,), jnp.float32),
        pltpu.VMEM((BPW,), jnp.float32),
        pltpu.SemaphoreType.DMA,
        pltpu.SemaphoreType.DMA,
    ],
    compiler_params=pltpu.CompilerParams(needs_layout_passes=False,
                                         use_tc_tiling_on_sc=True,
                                         disable_bounds_checks=True),
)(_svdpp_body)


def kernel(scientist_ids, paper_ids, scientist_factors, paper_factors,
           implicit_factors, scientist_bias, paper_bias, global_bias):
    del implicit_factors
    sids = scientist_ids.astype(jnp.int32).reshape(NW * NCH, 128)
    pids = paper_ids.astype(jnp.int32).reshape(NW * NCH, 128)
    sfac = _tstage(scientist_factors.T)
    pfac = _tstage(paper_factors.T)
    sb = scientist_bias.reshape(-1)
    pb = paper_bias.reshape(-1)
    g16 = jnp.broadcast_to(global_bias.astype(jnp.float32).reshape(()), (L,))
    return _svdpp(sids, pids, sfac, pfac, sb, pb, g16)


# speedup vs baseline: 1.3035x; 1.3035x over previous
import functools

import jax
import jax.numpy as jnp
from jax import lax
from jax.experimental import pallas as pl
from jax.experimental.pallas import tpu as pltpu
from jax.experimental.pallas import tpu_sc as plsc

NC = 2
NS = 16
L = 16
NW = NC * NS
B = 16384
D = 64
N = 100000
BPW = B // NW
NCH = BPW // 128
HALF = BPW // 2
CPH = HALF // L
SR = N * D // 128
NBLK = N // 128
BPT = (NBLK + NW - 1) // NW


TW = 8192
TH = TW // 2
NTB = (N + TW - 1) // TW
SRP = NTB * TH


def _tstage_body(inT_ref, out_ref):
    x = inT_ref[...]
    out_ref[:, 0:D] = x[:, 0:TH].T
    out_ref[:, D:128] = x[:, TH:TW].T


def _tstage(tableT):
    return pl.pallas_call(
        _tstage_body,
        grid=(NTB,),
        in_specs=[pl.BlockSpec((D, TW), lambda j: (0, j))],
        out_specs=pl.BlockSpec((TH, 128), lambda j: (j, 0)),
        out_shape=jax.ShapeDtypeStruct((SRP, 128), jnp.float32),
    )(tableT)


def _svdpp_body(sids_h, pids_h, sfac_h, pfac_h, sbias_h, pbias_h, g_h, out_h,
                sidx_v, pidx_v, sdidx_v, pdidx_v, sh_v, ph_v,
                srows_v, prows_v, sb_v, pb_v, g_v, out_v, sem, semb):
    w = lax.axis_index("s") * NC + lax.axis_index("c")
    base = w * BPW

    pltpu.sync_copy(sids_h.at[pl.ds(w * NCH, NCH)], sidx_v)
    pltpu.sync_copy(pids_h.at[pl.ds(w * NCH, NCH)], pidx_v)
    pltpu.sync_copy(g_h, g_v)

    bcps = []
    for c in range(NCH):
        r = pl.ds(c * 128, 128)
        bcps.append(pltpu.async_copy(sbias_h.at[sidx_v.at[c]], sb_v.at[r], semb))
        bcps.append(pltpu.async_copy(pbias_h.at[pidx_v.at[c]], pb_v.at[r], semb))

    sh1 = TW.bit_length() - 1
    sh2 = sh1 - 1

    def _didx(v):
        return lax.shift_left(lax.shift_right_logical(v, sh1), sh2) + \
            lax.bitwise_and(v, TW // 2 - 1)

    def _hoff(v):
        return lax.shift_left(
            lax.bitwise_and(lax.shift_right_logical(v, sh2), 1), 6)

    for c in range(NCH):
        for j in range(128 // L):
            sl = pl.ds(j * L, L)
            fl = pl.ds(c * 128 + j * L, L)
            sv = sidx_v[c, sl]
            pv = pidx_v[c, sl]
            sdidx_v[c, sl] = _didx(sv)
            pdidx_v[c, sl] = _didx(pv)
            sh_v[fl] = _hoff(sv)
            ph_v[fl] = _hoff(pv)

    iota = lax.broadcasted_iota(jnp.int32, (L,), 0)
    ones = jnp.ones((L,), jnp.int32)
    gvec = g_v[...]

    sbufs = (srows_v.at[pl.ds(0, 128)], srows_v.at[pl.ds(128, 128)])
    pbufs = (prows_v.at[pl.ds(0, 128)], prows_v.at[pl.ds(128, 128)])

    def fire(q):
        return [pltpu.async_copy(sfac_h.at[sdidx_v.at[q]], sbufs[q % 2], sem),
                pltpu.async_copy(pfac_h.at[pdidx_v.at[q]], pbufs[q % 2], sem)]

    pend = fire(0)
    for q in range(NCH):
        if q + 1 < NCH:
            nxt = fire(q + 1)
        for cp in pend:
            cp.wait()
        if q == 0:
            for cp in bcps:
                cp.wait()
        def chunk(i, carry, q=q):
            lrow = (q % 2) * 128 + (i - q * 8) * L + iota
            sh0 = sh_v[pl.ds(i * L, L)]
            ph0 = ph_v[pl.ds(i * L, L)]
            z = jnp.zeros((L,), jnp.float32)

            def dbody(t, st):
                a0, a1, a2, a3, mvec = st
                accs = [a0, a1, a2, a3]
                for k in range(8):
                    m = lax.bitwise_and(mvec + k if k else mvec, 63)
                    sv = plsc.load_gather(srows_v, [lrow, sh0 + m])
                    pv = plsc.load_gather(prows_v, [lrow, ph0 + m])
                    accs[k % 4] = accs[k % 4] + sv * pv
                return (*accs, mvec + 8)

            a0, a1, a2, a3, _ = lax.fori_loop(
                0, D // 8, dbody, (z, z, z, z, iota))
            sl = pl.ds(i * L, L)
            out_v[sl] = ((a0 + a1) + (a2 + a3)) + sb_v[sl] + pb_v[sl] + gvec
            return carry

        lax.fori_loop(q * 8, (q + 1) * 8, chunk, 0)
        if q + 1 < NCH:
            pend = nxt

    pltpu.sync_copy(out_v, out_h.at[pl.ds(base, BPW)])


_svdpp = functools.partial(
    pl.kernel,
    out_type=jax.ShapeDtypeStruct((B,), jnp.float32),
    mesh=plsc.VectorSubcoreMesh(core_axis_name="c", subcore_axis_name="s"),
    scratch_types=[
        pltpu.VMEM((NCH, 128), jnp.int32),
        pltpu.VMEM((NCH, 128), jnp.int32),
        pltpu.VMEM((NCH, 128), jnp.int32),
        pltpu.VMEM((NCH, 128), jnp.int32),
        pltpu.VMEM((BPW,), jnp.int32),
        pltpu.VMEM((BPW,), jnp.int32),
        pltpu.VMEM((HALF, 128), jnp.float32),
        pltpu.VMEM((HALF, 128), jnp.float32),
        pltpu.VMEM((BPW,), jnp.float32),
        pltpu.VMEM((BPW,), jnp.float32),
        pltpu.VMEM((L,), jnp.float32),
        pltpu.VMEM((BPW,), jnp.float32),
        pltpu.SemaphoreType.DMA,
        pltpu.SemaphoreType.DMA,
    ],
    compiler_params=pltpu.CompilerParams(needs_layout_passes=False,
                                         use_tc_tiling_on_sc=True,
                                         disable_bounds_checks=True),
)(_svdpp_body)


def kernel(scientist_ids, paper_ids, scientist_factors, paper_factors,
           implicit_factors, scientist_bias, paper_bias, global_bias):
    del implicit_factors
    sids = scientist_ids.astype(jnp.int32).reshape(NW * NCH, 128)
    pids = paper_ids.astype(jnp.int32).reshape(NW * NCH, 128)
    sfac = _tstage(scientist_factors.T)
    pfac = _tstage(paper_factors.T)
    sb = scientist_bias.reshape(-1)
    pb = paper_bias.reshape(-1)
    g16 = jnp.broadcast_to(global_bias.astype(jnp.float32).reshape(()), (L,))
    return _svdpp(sids, pids, sfac, pfac, sb, pb, g16)


# speedup vs baseline: 1.4632x; 1.1225x over previous
import functools

import jax
import jax.numpy as jnp
from jax import lax
from jax.experimental import pallas as pl
from jax.experimental.pallas import tpu as pltpu
from jax.experimental.pallas import tpu_sc as plsc

NC = 2
NS = 16
L = 16
NW = NC * NS
B = 16384
D = 64
N = 100000
BPW = B // NW
NCH = BPW // 128
HALF = BPW // 2
CPH = HALF // L
SR = N * D // 128
NBLK = N // 128
BPT = (NBLK + NW - 1) // NW


TW = 8192
TH = TW // 2
NTB = (N + TW - 1) // TW
SRP = NTB * TH


def _tstage_body(sT_ref, pT_ref, s_out_ref, p_out_ref):
    for src, dst in ((sT_ref, s_out_ref), (pT_ref, p_out_ref)):
        x = src[...]
        dst[:, 0:D] = x[:, 0:TH].T
        dst[:, D:128] = x[:, TH:TW].T


def _tstage(sT, pT):
    spec_in = pl.BlockSpec((D, TW), lambda j: (0, j))
    spec_out = pl.BlockSpec((TH, 128), lambda j: (j, 0))
    return pl.pallas_call(
        _tstage_body,
        grid=(NTB,),
        in_specs=[spec_in, spec_in],
        out_specs=[spec_out, spec_out],
        out_shape=(jax.ShapeDtypeStruct((SRP, 128), jnp.float32),
                   jax.ShapeDtypeStruct((SRP, 128), jnp.float32)),
    )(sT, pT)


def _svdpp_body(sids_h, pids_h, sfac_h, pfac_h, sbias_h, pbias_h, g_h, out_h,
                sidx_v, pidx_v, sdidx_v, pdidx_v, sh_v, ph_v,
                srows_v, prows_v, sb_v, pb_v, g_v, out_v, sem, semb):
    w = lax.axis_index("s") * NC + lax.axis_index("c")
    base = w * BPW

    pltpu.sync_copy(sids_h.at[pl.ds(w * NCH, NCH)], sidx_v)
    pltpu.sync_copy(pids_h.at[pl.ds(w * NCH, NCH)], pidx_v)
    pltpu.sync_copy(g_h, g_v)

    bcps = []
    for c in range(NCH):
        r = pl.ds(c * 128, 128)
        bcps.append(pltpu.async_copy(sbias_h.at[sidx_v.at[c]], sb_v.at[r], semb))
        bcps.append(pltpu.async_copy(pbias_h.at[pidx_v.at[c]], pb_v.at[r], semb))

    sh1 = TW.bit_length() - 1
    sh2 = sh1 - 1

    def _didx(v):
        return lax.shift_left(lax.shift_right_logical(v, sh1), sh2) + \
            lax.bitwise_and(v, TW // 2 - 1)

    def _hoff(v):
        return lax.shift_left(
            lax.bitwise_and(lax.shift_right_logical(v, sh2), 1), 6)

    for c in range(NCH):
        for j in range(128 // L):
            sl = pl.ds(j * L, L)
            fl = pl.ds(c * 128 + j * L, L)
            sv = sidx_v[c, sl]
            pv = pidx_v[c, sl]
            sdidx_v[c, sl] = _didx(sv)
            pdidx_v[c, sl] = _didx(pv)
            sh_v[fl] = _hoff(sv)
            ph_v[fl] = _hoff(pv)

    iota = lax.broadcasted_iota(jnp.int32, (L,), 0)
    ones = jnp.ones((L,), jnp.int32)
    gvec = g_v[...]

    sbufs = (srows_v.at[pl.ds(0, 128)], srows_v.at[pl.ds(128, 128)])
    pbufs = (prows_v.at[pl.ds(0, 128)], prows_v.at[pl.ds(128, 128)])

    def fire(q):
        return [pltpu.async_copy(sfac_h.at[sdidx_v.at[q]], sbufs[q % 2], sem),
                pltpu.async_copy(pfac_h.at[pdidx_v.at[q]], pbufs[q % 2], sem)]

    pend = fire(0)
    for q in range(NCH):
        if q + 1 < NCH:
            nxt = fire(q + 1)
        for cp in pend:
            cp.wait()
        if q == 0:
            for cp in bcps:
                cp.wait()
        def chunk(i, carry, q=q):
            lrow = (q % 2) * 128 + (i - q * 8) * L + iota
            sh0 = sh_v[pl.ds(i * L, L)]
            ph0 = ph_v[pl.ds(i * L, L)]
            z = jnp.zeros((L,), jnp.float32)

            def dbody(t, st):
                a0, a1, a2, a3, mvec = st
                accs = [a0, a1, a2, a3]
                for k in range(8):
                    m = lax.bitwise_and(mvec + k if k else mvec, 63)
                    sv = plsc.load_gather(srows_v, [lrow, sh0 + m])
                    pv = plsc.load_gather(prows_v, [lrow, ph0 + m])
                    accs[k % 4] = accs[k % 4] + sv * pv
                return (*accs, mvec + 8)

            a0, a1, a2, a3, _ = lax.fori_loop(
                0, D // 8, dbody, (z, z, z, z, iota))
            sl = pl.ds(i * L, L)
            out_v[sl] = ((a0 + a1) + (a2 + a3)) + sb_v[sl] + pb_v[sl] + gvec
            return carry

        lax.fori_loop(q * 8, (q + 1) * 8, chunk, 0)
        if q + 1 < NCH:
            pend = nxt

    pltpu.sync_copy(out_v, out_h.at[pl.ds(base, BPW)])


_svdpp = functools.partial(
    pl.kernel,
    out_type=jax.ShapeDtypeStruct((B,), jnp.float32),
    mesh=plsc.VectorSubcoreMesh(core_axis_name="c", subcore_axis_name="s"),
    scratch_types=[
        pltpu.VMEM((NCH, 128), jnp.int32),
        pltpu.VMEM((NCH, 128), jnp.int32),
        pltpu.VMEM((NCH, 128), jnp.int32),
        pltpu.VMEM((NCH, 128), jnp.int32),
        pltpu.VMEM((BPW,), jnp.int32),
        pltpu.VMEM((BPW,), jnp.int32),
        pltpu.VMEM((HALF, 128), jnp.float32),
        pltpu.VMEM((HALF, 128), jnp.float32),
        pltpu.VMEM((BPW,), jnp.float32),
        pltpu.VMEM((BPW,), jnp.float32),
        pltpu.VMEM((L,), jnp.float32),
        pltpu.VMEM((BPW,), jnp.float32),
        pltpu.SemaphoreType.DMA,
        pltpu.SemaphoreType.DMA,
    ],
    compiler_params=pltpu.CompilerParams(needs_layout_passes=False,
                                         use_tc_tiling_on_sc=True,
                                         disable_bounds_checks=True),
)(_svdpp_body)


def kernel(scientist_ids, paper_ids, scientist_factors, paper_factors,
           implicit_factors, scientist_bias, paper_bias, global_bias):
    del implicit_factors
    sids = scientist_ids.astype(jnp.int32).reshape(NW * NCH, 128)
    pids = paper_ids.astype(jnp.int32).reshape(NW * NCH, 128)
    sfac, pfac = _tstage(scientist_factors.T, paper_factors.T)
    sb = scientist_bias.reshape(-1)
    pb = paper_bias.reshape(-1)
    g16 = jnp.broadcast_to(global_bias.astype(jnp.float32).reshape(()), (L,))
    return _svdpp(sids, pids, sfac, pfac, sb, pb, g16)
